# baseline (speedup 1.0000x reference)
import jax
import jax.numpy as jnp
from jax import lax
from jax.experimental import pallas as pl
from jax.experimental.pallas import tpu as pltpu
from jax.experimental.pallas import tpu_sc as plsc

N = 99999
E = 6399936
EPAD = 6400000
PER_W = 200000
CHUNK = 4000
NCHUNK = PER_W // CHUNK
NVEC = CHUNK // 16
UNROLL = 8
NPACK = 50000
NOUT = 33333
ACC = 33536
COLS = ACC // 16
NCV = COLS // 16
CPAD = CHUNK + 16


def _sc_body(xp_hbm, src_hbm, dst_hbm, w_hbm, scr_hbm, part_hbm,
             xp_v, src_v, dst_v, qc_v, vc_v, acc_v, w_v, out_v):
    cid = lax.axis_index("c")
    sid = lax.axis_index("s")
    wid = sid * 2 + cid

    pltpu.sync_copy(xp_hbm, xp_v)
    pltpu.sync_copy(w_hbm, w_v)

    zero = jnp.zeros((16,), jnp.float32)
    iota = jax.lax.iota(jnp.int32, 16)

    @plsc.parallel_loop(0, COLS, unroll=8)
    def _zero(j):
        acc_v[pl.ds(j * 16, 16)] = zero

    ebase = wid * PER_W

    def chunk_body(g, carry):
        base = ebase + g * CHUNK
        pltpu.sync_copy(src_hbm.at[pl.ds(base, CHUNK)], src_v)
        pltpu.sync_copy(dst_hbm.at[pl.ds(base, CHUNK)], dst_v)

        @plsc.parallel_loop(0, NVEC, unroll=UNROLL, carry=jnp.int32(0))
        def pass1(i, cnt):
            o = i * 16
            s16 = src_v[pl.ds(o, 16)]
            d16 = dst_v[pl.ds(o, 16)]
            pk = plsc.load_gather(xp_v, [s16 >> 1])
            bits = jnp.where((s16 & 1) == 1, pk & jnp.int32(-65536),
                             pk << 16)
            val = plsc.bitcast(bits, jnp.float32)
            q = lax.div(d16, jnp.int32(3))
            msk = (d16 - q * 3) == 0
            plsc.store_compressed(qc_v.at[pl.ds(cnt, 16)], q, mask=msk)
            plsc.store_compressed(vc_v.at[pl.ds(cnt, 16)], val, mask=msk)
            return cnt + jnp.max(plsc.all_reduce_population_count(msk))

        cnt = pass1
        qc_v[pl.ds(cnt, 16)] = iota
        vc_v[pl.ds(cnt, 16)] = zero
        nvc = (cnt + 15) >> 4

        def pass2(k, c2):
            o = k * 16
            qv = qc_v[pl.ds(o, 16)]
            vv = vc_v[pl.ds(o, 16)]
            plsc.addupdate_scatter(acc_v, [qv], vv)
            return c2
        lax.fori_loop(0, nvc, pass2, 0)
        return carry
    lax.fori_loop(0, NCHUNK, chunk_body, 0)

    pltpu.sync_copy(acc_v, scr_hbm.at[pl.ds((cid * 16 + sid) * ACC, ACC)])
    plsc.subcore_barrier()

    colbase = sid * COLS
    for p in range(16):
        pltpu.sync_copy(scr_hbm.at[pl.ds((cid * 16 + p) * ACC + colbase,
                                         COLS)],
                        acc_v.at[pl.ds(p * COLS, COLS)])

    wv = w_v[...]

    @plsc.parallel_loop(0, NCV, unroll=2)
    def rbody(j):
        o = j * 16
        t = acc_v[pl.ds(o, 16)]
        for p in range(1, 16):
            t = t + acc_v[pl.ds(p * COLS + o, 16)]
        out_v[pl.ds(o, 16)] = t * wv

    pltpu.sync_copy(out_v, part_hbm.at[pl.ds(cid * ACC + colbase, COLS)])


def _combine_body(p_ref, o_ref):
    o_ref[...] = p_ref[:ACC] + p_ref[ACC:]


def kernel(x, edge_index, W):
    xb = x.reshape(-1).astype(jnp.bfloat16)
    xb = jnp.concatenate([xb, jnp.zeros((1,), jnp.bfloat16)])
    xp = lax.bitcast_convert_type(xb.reshape(NPACK, 2), jnp.int32)

    pad = EPAD - E
    src = jnp.concatenate([edge_index[0], jnp.zeros((pad,), jnp.int32)])
    dst = jnp.concatenate([edge_index[1], jnp.ones((pad,), jnp.int32)])
    wvec = jnp.broadcast_to(W.reshape(()), (16,)).astype(jnp.float32)

    mesh = plsc.VectorSubcoreMesh(core_axis_name="c", subcore_axis_name="s",
                                  num_cores=2, num_subcores=16)
    _, part = pl.kernel(
        _sc_body,
        out_type=(jax.ShapeDtypeStruct((32 * ACC,), jnp.float32),
                  jax.ShapeDtypeStruct((2 * ACC,), jnp.float32)),
        mesh=mesh,
        compiler_params=pltpu.CompilerParams(needs_layout_passes=False),
        scratch_types=[
            pltpu.VMEM((NPACK,), jnp.int32),
            pltpu.VMEM((CHUNK,), jnp.int32),
            pltpu.VMEM((CHUNK,), jnp.int32),
            pltpu.VMEM((CPAD,), jnp.int32),
            pltpu.VMEM((CPAD,), jnp.float32),
            pltpu.VMEM((ACC,), jnp.float32),
            pltpu.VMEM((16,), jnp.float32),
            pltpu.VMEM((COLS,), jnp.float32),
        ],
    )(xp, src, dst, wvec)

    out = pl.pallas_call(
        _combine_body,
        out_shape=jax.ShapeDtypeStruct((ACC,), jnp.float32),
    )(part)
    return out[:NOUT]


# speedup vs baseline: 1.1322x; 1.1322x over previous
import jax
import jax.numpy as jnp
from jax import lax
from jax.experimental import pallas as pl
from jax.experimental.pallas import tpu as pltpu
from jax.experimental.pallas import tpu_sc as plsc

N = 99999
E = 6399936
EPAD = 6400000
PER_W = 200000
CHUNK = 4000
NCHUNK = PER_W // CHUNK
NVEC = CHUNK // 16
UNROLL = 8
NPACK = 50000
NOUT = 33333
ACC = 33536
COLS = ACC // 16
NCV = COLS // 16
CPAD = CHUNK + 16


def _sc_body(xp_hbm, src_hbm, dst_hbm, w_hbm, scr_hbm, part_hbm,
             xp_v, src_v, dst_v, acc_v, acc_b, w_v, out_v):
    cid = lax.axis_index("c")
    sid = lax.axis_index("s")
    wid = sid * 2 + cid

    pltpu.sync_copy(xp_hbm, xp_v)
    pltpu.sync_copy(w_hbm, w_v)

    zero = jnp.zeros((16,), jnp.float32)
    iota = jax.lax.iota(jnp.int32, 16)

    @plsc.parallel_loop(0, COLS, unroll=8)
    def _zero(j):
        acc_v[pl.ds(j * 16, 16)] = zero
        acc_b[pl.ds(j * 16, 16)] = zero

    ebase = wid * PER_W

    def chunk_body(g, carry):
        base = ebase + g * CHUNK
        pltpu.sync_copy(src_hbm.at[pl.ds(base, CHUNK)], src_v)
        pltpu.sync_copy(dst_hbm.at[pl.ds(base, CHUNK)], dst_v)

        @plsc.parallel_loop(0, NVEC // 2, unroll=UNROLL)
        def inner(i):
            for u, acc in ((0, acc_v), (1, acc_b)):
                o = (i * 2 + u) * 16
                s16 = src_v[pl.ds(o, 16)]
                d16 = dst_v[pl.ds(o, 16)]
                pk = plsc.load_gather(xp_v, [s16 >> 1])
                bits = jnp.where((s16 & 1) == 1, pk & jnp.int32(-65536),
                                 pk << 16)
                val = plsc.bitcast(bits, jnp.float32)
                q = lax.div(d16, jnp.int32(3))
                val = jnp.where((d16 - q * 3) == 0, val, 0.0)
                plsc.addupdate_scatter(acc, [q], val)
        return carry
    lax.fori_loop(0, NCHUNK, chunk_body, 0)

    @plsc.parallel_loop(0, ACC // 16, unroll=8)
    def _merge(j):
        o = j * 16
        acc_v[pl.ds(o, 16)] = acc_v[pl.ds(o, 16)] + acc_b[pl.ds(o, 16)]

    pltpu.sync_copy(acc_v, scr_hbm.at[pl.ds((cid * 16 + sid) * ACC, ACC)])
    plsc.subcore_barrier()

    colbase = sid * COLS
    for p in range(16):
        pltpu.sync_copy(scr_hbm.at[pl.ds((cid * 16 + p) * ACC + colbase,
                                         COLS)],
                        acc_v.at[pl.ds(p * COLS, COLS)])

    wv = w_v[...]

    @plsc.parallel_loop(0, NCV, unroll=2)
    def rbody(j):
        o = j * 16
        t = acc_v[pl.ds(o, 16)]
        for p in range(1, 16):
            t = t + acc_v[pl.ds(p * COLS + o, 16)]
        out_v[pl.ds(o, 16)] = t * wv

    pltpu.sync_copy(out_v, part_hbm.at[pl.ds(cid * ACC + colbase, COLS)])


def _combine_body(p_ref, o_ref):
    o_ref[...] = p_ref[:ACC] + p_ref[ACC:]


def kernel(x, edge_index, W):
    xb = x.reshape(-1).astype(jnp.bfloat16)
    xb = jnp.concatenate([xb, jnp.zeros((1,), jnp.bfloat16)])
    xp = lax.bitcast_convert_type(xb.reshape(NPACK, 2), jnp.int32)

    pad = EPAD - E
    src = jnp.concatenate([edge_index[0], jnp.zeros((pad,), jnp.int32)])
    dst = jnp.concatenate([edge_index[1], jnp.ones((pad,), jnp.int32)])
    wvec = jnp.broadcast_to(W.reshape(()), (16,)).astype(jnp.float32)

    mesh = plsc.VectorSubcoreMesh(core_axis_name="c", subcore_axis_name="s",
                                  num_cores=2, num_subcores=16)
    _, part = pl.kernel(
        _sc_body,
        out_type=(jax.ShapeDtypeStruct((32 * ACC,), jnp.float32),
                  jax.ShapeDtypeStruct((2 * ACC,), jnp.float32)),
        mesh=mesh,
        compiler_params=pltpu.CompilerParams(needs_layout_passes=False),
        scratch_types=[
            pltpu.VMEM((NPACK,), jnp.int32),
            pltpu.VMEM((CHUNK,), jnp.int32),
            pltpu.VMEM((CHUNK,), jnp.int32),
            pltpu.VMEM((ACC,), jnp.float32),
            pltpu.VMEM((ACC,), jnp.float32),
            pltpu.VMEM((16,), jnp.float32),
            pltpu.VMEM((COLS,), jnp.float32),
        ],
    )(xp, src, dst, wvec)

    out = pl.pallas_call(
        _combine_body,
        out_shape=jax.ShapeDtypeStruct((ACC,), jnp.float32),
    )(part)
    return out[:NOUT]


# speedup vs baseline: 1.3839x; 1.2222x over previous
import jax
import jax.numpy as jnp
from jax import lax
from jax.experimental import pallas as pl
from jax.experimental.pallas import tpu as pltpu
from jax.experimental.pallas import tpu_sc as plsc

N = 99999
E = 6399936
PER_W = 200000
LAST_W = E - 31 * PER_W
CHUNK = 4000
NCHUNK = PER_W // CHUNK
LAST_FULL = LAST_W // CHUNK
LAST_REM = LAST_W - LAST_FULL * CHUNK
NVEC = CHUNK // 16
UNROLL = 8
NPACK = 50000
NOUT = 33333
ACC = 33536
COLS = ACC // 16
NCV = COLS // 16


def _sc_body(xp_hbm, src_hbm, dst_hbm, w_hbm, part_hbm,
             xp_v, src_v, dst_v, acc_v, w_v, out_v, shr):
    cid = lax.axis_index("c")
    sid = lax.axis_index("s")
    wid = sid * 2 + cid

    pltpu.sync_copy(xp_hbm, xp_v)
    pltpu.sync_copy(w_hbm, w_v)

    zero = jnp.zeros((16,), jnp.float32)

    @plsc.parallel_loop(0, COLS, unroll=8)
    def _zero(j):
        acc_v[pl.ds(j * 16, 16)] = zero

    ebase = wid * PER_W
    is_last = wid == 31

    def do_chunk(base, nvec):
        pltpu.sync_copy(src_hbm.at[pl.ds(base, nvec * 16)],
                        src_v.at[pl.ds(0, nvec * 16)])
        pltpu.sync_copy(dst_hbm.at[pl.ds(base, nvec * 16)],
                        dst_v.at[pl.ds(0, nvec * 16)])

        @plsc.parallel_loop(0, nvec, unroll=UNROLL)
        def inner(i):
            o = i * 16
            s16 = src_v[pl.ds(o, 16)]
            d16 = dst_v[pl.ds(o, 16)]
            pk = plsc.load_gather(xp_v, [s16 >> 1])
            bits = jnp.where((s16 & 1) == 1, pk & jnp.int32(-65536),
                             pk << 16)
            val = plsc.bitcast(bits, jnp.float32)
            q = lax.div(d16, jnp.int32(3))
            val = jnp.where((d16 - q * 3) == 0, val, 0.0)
            plsc.addupdate_scatter(acc_v, [q], val)

    def chunk_body(g, carry):
        do_chunk(ebase + g * CHUNK, NVEC)
        return carry
    nfull = jnp.where(is_last, LAST_FULL, NCHUNK)
    lax.fori_loop(0, nfull, chunk_body, 0)

    @pl.when(is_last)
    def _tail():
        do_chunk(ebase + LAST_FULL * CHUNK, LAST_REM // 16)

    pltpu.sync_copy(acc_v, shr.at[pl.ds(sid * ACC, ACC)])
    plsc.subcore_barrier()

    colbase = sid * COLS
    for p in range(16):
        pltpu.sync_copy(shr.at[pl.ds(p * ACC + colbase, COLS)],
                        acc_v.at[pl.ds(p * COLS, COLS)])

    wv = w_v[...]

    @plsc.parallel_loop(0, NCV, unroll=2)
    def rbody(j):
        o = j * 16
        t = acc_v[pl.ds(o, 16)]
        for p in range(1, 16):
            t = t + acc_v[pl.ds(p * COLS + o, 16)]
        out_v[pl.ds(o, 16)] = t * wv

    pltpu.sync_copy(out_v, part_hbm.at[pl.ds(cid * ACC + colbase, COLS)])


def _combine_body(p_ref, o_ref):
    o_ref[...] = p_ref[:ACC] + p_ref[ACC:]


def kernel(x, edge_index, W):
    xb = x.reshape(-1).astype(jnp.bfloat16)
    xb = jnp.concatenate([xb, jnp.zeros((1,), jnp.bfloat16)])
    xp = lax.bitcast_convert_type(xb.reshape(NPACK, 2), jnp.int32)

    src = edge_index[0]
    dst = edge_index[1]
    wvec = jnp.broadcast_to(W.reshape(()), (16,)).astype(jnp.float32)

    mesh = plsc.VectorSubcoreMesh(core_axis_name="c", subcore_axis_name="s",
                                  num_cores=2, num_subcores=16)
    part = pl.kernel(
        _sc_body,
        out_type=jax.ShapeDtypeStruct((2 * ACC,), jnp.float32),
        mesh=mesh,
        compiler_params=pltpu.CompilerParams(needs_layout_passes=False),
        scratch_types=[
            pltpu.VMEM((NPACK,), jnp.int32),
            pltpu.VMEM((CHUNK,), jnp.int32),
            pltpu.VMEM((CHUNK,), jnp.int32),
            pltpu.VMEM((ACC,), jnp.float32),
            pltpu.VMEM((16,), jnp.float32),
            pltpu.VMEM((COLS,), jnp.float32),
            pltpu.VMEM_SHARED((16 * ACC,), jnp.float32),
        ],
    )(xp, src, dst, wvec)

    out = pl.pallas_call(
        _combine_body,
        out_shape=jax.ShapeDtypeStruct((ACC,), jnp.float32),
    )(part)
    return out[:NOUT]


# speedup vs baseline: 1.4047x; 1.0151x over previous
import jax
import jax.numpy as jnp
from jax import lax
from jax.experimental import pallas as pl
from jax.experimental.pallas import tpu as pltpu
from jax.experimental.pallas import tpu_sc as plsc

N = 99999
E = 6399936
PER_W = 200000
LAST_W = E - 31 * PER_W
CHUNK = 2000
NCHUNK = PER_W // CHUNK
LAST_FULL = LAST_W // CHUNK
LAST_REM = LAST_W - LAST_FULL * CHUNK
NVEC = CHUNK // 16
UNROLL = 5
NPACK = 50000
NOUT = 33333
ACC = 33536
COLS = ACC // 16
NCV = COLS // 16


def _sc_body(xp_hbm, src_hbm, dst_hbm, w_hbm, part_hbm,
             xp_v, src_v, dst_v, src_w, dst_w, acc_v, w_v, out_v,
             sem0, sem1, shr):
    cid = lax.axis_index("c")
    sid = lax.axis_index("s")
    wid = sid * 2 + cid

    pltpu.sync_copy(xp_hbm, xp_v)
    pltpu.sync_copy(w_hbm, w_v)

    zero = jnp.zeros((16,), jnp.float32)

    @plsc.parallel_loop(0, COLS, unroll=8)
    def _zero(j):
        acc_v[pl.ds(j * 16, 16)] = zero

    ebase = wid * PER_W
    is_last = wid == 31

    def compute(sv, dv, nvec, unroll=UNROLL):
        @plsc.parallel_loop(0, nvec, unroll=unroll)
        def inner(i):
            o = i * 16
            s16 = sv[pl.ds(o, 16)]
            d16 = dv[pl.ds(o, 16)]
            pk = plsc.load_gather(xp_v, [s16 >> 1])
            bits = jnp.where((s16 & 1) == 1, pk & jnp.int32(-65536),
                             pk << 16)
            val = plsc.bitcast(bits, jnp.float32)
            q = lax.div(d16, jnp.int32(3))
            val = jnp.where((d16 - q * 3) == 0, val, 0.0)
            plsc.addupdate_scatter(acc_v, [q], val)

    def do_chunk(base, nvec, sv, dv, unroll):
        pltpu.sync_copy(src_hbm.at[pl.ds(base, nvec * 16)],
                        sv.at[pl.ds(0, nvec * 16)])
        pltpu.sync_copy(dst_hbm.at[pl.ds(base, nvec * 16)],
                        dv.at[pl.ds(0, nvec * 16)])
        compute(sv, dv, nvec, unroll)

    npair = jnp.where(is_last, LAST_FULL // 2, NCHUNK // 2)
    nchunks = npair * 2
    bufs = ((src_v, dst_v, sem0), (src_w, dst_w, sem1))

    def start(g, sv, dv, sem):
        base = ebase + g * CHUNK
        pltpu.async_copy(src_hbm.at[pl.ds(base, CHUNK)], sv, sem)
        pltpu.async_copy(dst_hbm.at[pl.ds(base, CHUNK)], dv, sem)

    def wait(g, sv, dv, sem):
        base = ebase + g * CHUNK
        pltpu.make_async_copy(src_hbm.at[pl.ds(base, CHUNK)], sv, sem).wait()
        pltpu.make_async_copy(dst_hbm.at[pl.ds(base, CHUNK)], dv, sem).wait()

    start(0, *bufs[0])
    start(1, *bufs[1])

    def pair_body(t, carry):
        for b, (sv, dv, sem) in enumerate(bufs):
            g = t * 2 + b
            wait(g, sv, dv, sem)
            compute(sv, dv, NVEC)

            @pl.when(g + 2 < nchunks)
            def _prefetch():
                start(g + 2, sv, dv, sem)
        return carry
    lax.fori_loop(0, npair, pair_body, 0)

    @pl.when(is_last)
    def _tail():
        do_chunk(ebase + (LAST_FULL - 1) * CHUNK, NVEC, src_v, dst_v, UNROLL)
        do_chunk(ebase + LAST_FULL * CHUNK, LAST_REM // 16, src_v, dst_v, 11)

    pltpu.sync_copy(acc_v, shr.at[pl.ds(sid * ACC, ACC)])
    plsc.subcore_barrier()

    colbase = sid * COLS
    for p in range(16):
        pltpu.sync_copy(shr.at[pl.ds(p * ACC + colbase, COLS)],
                        acc_v.at[pl.ds(p * COLS, COLS)])

    wv = w_v[...]

    @plsc.parallel_loop(0, NCV, unroll=2)
    def rbody(j):
        o = j * 16
        t = acc_v[pl.ds(o, 16)]
        for p in range(1, 16):
            t = t + acc_v[pl.ds(p * COLS + o, 16)]
        out_v[pl.ds(o, 16)] = t * wv

    pltpu.sync_copy(out_v, part_hbm.at[pl.ds(cid * ACC + colbase, COLS)])


def _combine_body(p_ref, o_ref):
    o_ref[...] = p_ref[:ACC] + p_ref[ACC:]


def kernel(x, edge_index, W):
    xb = x.reshape(-1).astype(jnp.bfloat16)
    xb = jnp.concatenate([xb, jnp.zeros((1,), jnp.bfloat16)])
    xp = lax.bitcast_convert_type(xb.reshape(NPACK, 2), jnp.int32)

    src = edge_index[0]
    dst = edge_index[1]
    wvec = jnp.broadcast_to(W.reshape(()), (16,)).astype(jnp.float32)

    mesh = plsc.VectorSubcoreMesh(core_axis_name="c", subcore_axis_name="s",
                                  num_cores=2, num_subcores=16)
    part = pl.kernel(
        _sc_body,
        out_type=jax.ShapeDtypeStruct((2 * ACC,), jnp.float32),
        mesh=mesh,
        compiler_params=pltpu.CompilerParams(needs_layout_passes=False),
        scratch_types=[
            pltpu.VMEM((NPACK,), jnp.int32),
            pltpu.VMEM((CHUNK,), jnp.int32),
            pltpu.VMEM((CHUNK,), jnp.int32),
            pltpu.VMEM((CHUNK,), jnp.int32),
            pltpu.VMEM((CHUNK,), jnp.int32),
            pltpu.VMEM((ACC,), jnp.float32),
            pltpu.VMEM((16,), jnp.float32),
            pltpu.VMEM((COLS,), jnp.float32),
            pltpu.SemaphoreType.DMA,
            pltpu.SemaphoreType.DMA,
            pltpu.VMEM_SHARED((16 * ACC,), jnp.float32),
        ],
    )(xp, src, dst, wvec)

    out = pl.pallas_call(
        _combine_body,
        out_shape=jax.ShapeDtypeStruct((ACC,), jnp.float32),
    )(part)
    return out[:NOUT]
